# Initial kernel scaffold; baseline (speedup 1.0000x reference)
import jax
import jax.numpy as jnp
from jax.experimental import pallas as pl


def kernel(x, edge_index, W_down, b_down, W1, b1, W2, b2, Wc, bc, Wd2, bd2, Wd3, bd3):
    raise NotImplementedError("write your pallas kernel here")


# speedup vs baseline: 10.3238x; 10.3238x over previous
import functools

import jax
import jax.numpy as jnp
from jax import lax
from jax.experimental import pallas as pl
from jax.experimental.pallas import tpu as pltpu
from jax.experimental.pallas import tpu_sc as plsc

N = 10000
D = 128
E = 320000
C = 40

NC, NS = 2, 16
NW = NC * NS
CH = 128
CPT = -(-E // (NW * CH))
EPW = CH * CPT
E_PAD = NW * EPW
RPT = 624
TAIL = N - NS * RPT
DUMP = N
N_ACC = N + 16
DEG_W = 16

_sc_mesh = plsc.VectorSubcoreMesh(
    core_axis_name="c", subcore_axis_name="s", num_cores=NC, num_subcores=NS)


@functools.partial(
    pl.kernel,
    out_type=jax.ShapeDtypeStruct((NC, N, DEG_W), jnp.float32),
    mesh=_sc_mesh,
    scratch_types=[
        pltpu.VMEM((CH,), jnp.int32),
        pltpu.VMEM((CH, DEG_W), jnp.float32),
        pltpu.VMEM_SHARED((N_ACC, DEG_W), jnp.float32),
    ],
)
def _sc_degrees(ones_hbm, dst_hbm, out_hbm, dst_v, ones_v, acc):
    cid = lax.axis_index("c")
    sid = lax.axis_index("s")
    base = (cid * NS + sid) * EPW
    r0 = sid * RPT
    pltpu.sync_copy(ones_hbm.at[pl.ds(0, RPT)], acc.at[pl.ds(r0, RPT)])

    @pl.when(sid == NS - 1)
    def _():
        pltpu.sync_copy(ones_hbm.at[pl.ds(0, TAIL)], acc.at[pl.ds(NS * RPT, TAIL)])

    pltpu.sync_copy(ones_hbm.at[pl.ds(0, CH)], ones_v)
    plsc.subcore_barrier()

    @pl.loop(0, CPT)
    def _(i):
        off = pl.multiple_of(base + i * CH, 8)
        pltpu.sync_copy(dst_hbm.at[pl.ds(off, CH)], dst_v)
        pltpu.sync_copy(ones_v, acc.at[dst_v], add=True)

    plsc.subcore_barrier()
    pltpu.sync_copy(acc.at[pl.ds(r0, RPT)], out_hbm.at[cid, pl.ds(r0, RPT)])

    @pl.when(sid == NS - 1)
    def _():
        pltpu.sync_copy(acc.at[pl.ds(NS * RPT, TAIL)],
                        out_hbm.at[cid, pl.ds(NS * RPT, TAIL)])


@functools.partial(
    pl.kernel,
    out_type=jax.ShapeDtypeStruct((NC, N, D), jnp.float32),
    mesh=_sc_mesh,
    scratch_types=[
        pltpu.VMEM((CH,), jnp.int32),
        pltpu.VMEM((CH,), jnp.int32),
        pltpu.VMEM((CH, D), jnp.float32),
        pltpu.VMEM_SHARED((N_ACC, D), jnp.float32),
        pltpu.SemaphoreType.DMA,
    ],
)
def _sc_edge_agg(u_hbm, src_hbm, dst_hbm, out_hbm, src_v, dst_v, rows_v, acc, sem):
    cid = lax.axis_index("c")
    sid = lax.axis_index("s")
    base = (cid * NS + sid) * EPW
    r0 = sid * RPT
    pltpu.sync_copy(u_hbm.at[pl.ds(r0, RPT)], acc.at[pl.ds(r0, RPT)])

    @pl.when(sid == NS - 1)
    def _():
        pltpu.sync_copy(u_hbm.at[pl.ds(NS * RPT, TAIL)],
                        acc.at[pl.ds(NS * RPT, TAIL)])

    plsc.subcore_barrier()

    @pl.loop(0, CPT)
    def _(i):
        off = pl.multiple_of(base + i * CH, 8)
        pltpu.sync_copy(src_hbm.at[pl.ds(off, CH)], src_v)
        pltpu.sync_copy(dst_hbm.at[pl.ds(off, CH)], dst_v)
        pltpu.async_copy(u_hbm.at[src_v], rows_v, sem).wait()
        pltpu.sync_copy(rows_v, acc.at[dst_v], add=True)

    plsc.subcore_barrier()
    pltpu.sync_copy(acc.at[pl.ds(r0, RPT)], out_hbm.at[cid, pl.ds(r0, RPT)])

    @pl.when(sid == NS - 1)
    def _():
        pltpu.sync_copy(acc.at[pl.ds(NS * RPT, TAIL)],
                        out_hbm.at[cid, pl.ds(NS * RPT, TAIL)])


BR = 1000


def _dinv(dg0_ref, dg1_ref):
    return lax.rsqrt(dg0_ref[:, :1] + dg1_ref[:, :1] - 1.0)


def _tc1_body(x_ref, w1_ref, wd_ref, bd_ref, dg0_ref, dg1_ref, u1_ref, ox_ref):
    x = x_ref[...]
    u1_ref[...] = _dinv(dg0_ref, dg1_ref) * jnp.dot(
        x, w1_ref[...], preferred_element_type=jnp.float32)
    ox_ref[...] = jnp.dot(
        x, wd_ref[...], preferred_element_type=jnp.float32) + bd_ref[...]


def _tc2_body(s0_ref, s1_ref, u1_ref, dg0_ref, dg1_ref, b1_ref, w2_ref, u2_ref):
    di = _dinv(dg0_ref, dg1_ref)
    h = di * (s0_ref[...] + s1_ref[...] - u1_ref[...]) + b1_ref[...]
    h = jnp.maximum(h, 0.0)
    u2_ref[...] = di * jnp.dot(h, w2_ref[...], preferred_element_type=jnp.float32)


def _tc3_body(s0_ref, s1_ref, u2_ref, dg0_ref, dg1_ref, b2_ref, ox_ref,
              wh_ref, bh_ref, o1_ref, o2_ref, o3_ref):
    di = _dinv(dg0_ref, dg1_ref)
    h = di * (s0_ref[...] + s1_ref[...] - u2_ref[...]) + b2_ref[...] + ox_ref[...]
    z = jnp.dot(h, wh_ref[...], preferred_element_type=jnp.float32) + bh_ref[...]
    zc = z[:, :C]
    m = jnp.max(zc, axis=1, keepdims=True)
    o1_ref[...] = zc - m - jnp.log(jnp.sum(jnp.exp(zc - m), axis=1, keepdims=True))
    o2_ref[...] = z[:, C:C + 1]
    o3_ref[...] = z[:, C + 1:C + 2]


def _row_spec(w):
    return pl.BlockSpec((BR, w), lambda i: (i, 0))


def _const_spec(shape):
    return pl.BlockSpec(shape, lambda i: (0,) * len(shape))


def kernel(x, edge_index, W_down, b_down, W1, b1, W2, b2, Wc, bc, Wd2, bd2, Wd3, bd3):
    pad = E_PAD - E
    src_p = jnp.concatenate([edge_index[0], jnp.zeros((pad,), jnp.int32)])
    dst_p = jnp.concatenate([edge_index[1], jnp.full((pad,), DUMP, jnp.int32)])
    ones = jnp.ones((RPT, DEG_W), jnp.float32)

    degp = _sc_degrees(ones, dst_p)
    dg0, dg1 = degp[0], degp[1]

    grid = (N // BR,)
    u1, ox = pl.pallas_call(
        _tc1_body,
        grid=grid,
        in_specs=[_row_spec(D), _const_spec((D, D)), _const_spec((D, D)),
                  _const_spec((1, D)), _row_spec(DEG_W), _row_spec(DEG_W)],
        out_specs=[_row_spec(D), _row_spec(D)],
        out_shape=[jax.ShapeDtypeStruct((N, D), jnp.float32)] * 2,
    )(x, W1, W_down, b_down.reshape(1, D), dg0, dg1)

    s1 = _sc_edge_agg(u1, src_p, dst_p)

    u2 = pl.pallas_call(
        _tc2_body,
        grid=grid,
        in_specs=[_row_spec(D), _row_spec(D), _row_spec(D), _row_spec(DEG_W),
                  _row_spec(DEG_W), _const_spec((1, D)), _const_spec((D, D))],
        out_specs=_row_spec(D),
        out_shape=jax.ShapeDtypeStruct((N, D), jnp.float32),
    )(s1[0], s1[1], u1, dg0, dg1, b1.reshape(1, D), W2)

    s2 = _sc_edge_agg(u2, src_p, dst_p)

    Wh = jnp.concatenate([Wc, Wd2, Wd3], axis=1)
    bh = jnp.concatenate([bc, bd2, bd3]).reshape(1, C + 2)
    o1, o2, o3 = pl.pallas_call(
        _tc3_body,
        grid=grid,
        in_specs=[_row_spec(D), _row_spec(D), _row_spec(D), _row_spec(DEG_W),
                  _row_spec(DEG_W), _const_spec((1, D)), _row_spec(D),
                  _const_spec((D, C + 2)), _const_spec((1, C + 2))],
        out_specs=[_row_spec(C), _row_spec(1), _row_spec(1)],
        out_shape=[jax.ShapeDtypeStruct((N, C), jnp.float32),
                   jax.ShapeDtypeStruct((N, 1), jnp.float32),
                   jax.ShapeDtypeStruct((N, 1), jnp.float32)],
    )(s2[0], s2[1], u2, dg0, dg1, b2.reshape(1, D), ox, Wh, bh)

    return (o1, jnp.squeeze(o2, -1), jnp.squeeze(o3, -1))
